# Initial kernel scaffold; baseline (speedup 1.0000x reference)
import functools

import jax
import jax.numpy as jnp
from jax import lax
from jax.experimental import pallas as pl
from jax.experimental.pallas import tpu as pltpu
from jax.experimental.pallas import tpu_sc as plsc

N = 10000
D = 128
NE = 320000
NC = 2
NS = 16
NW = NC * NS
EPT = NE // NW
C = 80
NCHUNK = EPT // C
RPT = N // NS

_mesh = plsc.VectorSubcoreMesh(core_axis_name="c", subcore_axis_name="s")


@functools.partial(
    pl.kernel,
    out_type=jax.ShapeDtypeStruct((NC, N, D), jnp.float32),
    mesh=_mesh,
    scratch_types=[
        pltpu.VMEM((C,), jnp.int32),
        pltpu.VMEM((C,), jnp.int32),
        pltpu.VMEM((C, D), jnp.float32),
        pltpu.VMEM_SHARED((N, D), jnp.float32),
        pltpu.SemaphoreType.DMA,
    ],
)
def _seg_sum(x_hbm, src_hbm, dst_hbm, out_hbm, src_v, dst_v, rows_v, acc, sem):
    cid = lax.axis_index("c")
    sid = lax.axis_index("s")
    wid = cid * NS + sid

    def zfill(i, carry):
        for t in range(D // 16):
            rows_v[i, pl.ds(t * 16, 16)] = jnp.zeros((16,), jnp.float32)
        return carry

    lax.fori_loop(0, C, zfill, 0)

    row0 = sid * RPT

    def zcopy(k, carry):
        pltpu.sync_copy(rows_v, acc.at[pl.ds(row0 + k * C, C), :])
        return carry

    lax.fori_loop(0, RPT // C, zcopy, 0)
    rem = RPT - (RPT // C) * C
    pltpu.sync_copy(rows_v.at[pl.ds(0, rem)],
                    acc.at[pl.ds(row0 + (RPT // C) * C, rem), :])
    plsc.subcore_barrier()

    ebase = wid * EPT

    def chunk(j, carry):
        off = ebase + j * C
        pltpu.sync_copy(src_hbm.at[pl.ds(off, C)], src_v)
        pltpu.sync_copy(dst_hbm.at[pl.ds(off, C)], dst_v)
        pltpu.async_copy(x_hbm.at[src_v], rows_v, sem).wait()
        pltpu.sync_copy(rows_v, acc.at[dst_v], add=True)
        return carry

    lax.fori_loop(0, NCHUNK, chunk, 0)
    plsc.subcore_barrier()

    pltpu.sync_copy(acc.at[pl.ds(row0, RPT), :],
                    out_hbm.at[cid, pl.ds(row0, RPT), :])


@functools.partial(
    pl.kernel,
    out_type=jax.ShapeDtypeStruct((NE,), jnp.float32),
    mesh=_mesh,
    scratch_types=[
        pltpu.VMEM((C,), jnp.int32),
        pltpu.VMEM((C,), jnp.int32),
        pltpu.VMEM((C, D), jnp.float32),
        pltpu.VMEM((C, D), jnp.float32),
        pltpu.VMEM((C,), jnp.float32),
        pltpu.SemaphoreType.DMA,
        pltpu.SemaphoreType.DMA,
    ],
)
def _edge_dot(h_hbm, e0_hbm, e1_hbm, out_hbm, i0_v, i1_v, r0_v, r1_v, o_v,
              sem0, sem1):
    cid = lax.axis_index("c")
    sid = lax.axis_index("s")
    wid = cid * NS + sid
    ebase = wid * EPT
    lane = lax.iota(jnp.int32, 16)

    def chunk(j, carry):
        off = ebase + j * C
        pltpu.sync_copy(e0_hbm.at[pl.ds(off, C)], i0_v)
        pltpu.sync_copy(e1_hbm.at[pl.ds(off, C)], i1_v)
        d0 = pltpu.async_copy(h_hbm.at[i0_v], r0_v, sem0)
        d1 = pltpu.async_copy(h_hbm.at[i1_v], r1_v, sem1)
        d0.wait()
        d1.wait()

        def group(g, carry2):
            rows = g * 16 + lane

            def dcol(d, acc):
                cols = jnp.full((16,), d, jnp.int32)
                a = plsc.load_gather(r0_v, [rows, cols])
                b = plsc.load_gather(r1_v, [rows, cols])
                return acc + a * b

            acc = lax.fori_loop(0, D, dcol, jnp.zeros((16,), jnp.float32))
            o_v[pl.ds(g * 16, 16)] = 1.0 / (1.0 + jnp.exp(-acc))
            return carry2

        lax.fori_loop(0, C // 16, group, 0)
        pltpu.sync_copy(o_v, out_hbm.at[pl.ds(off, C)])
        return carry

    lax.fori_loop(0, NCHUNK, chunk, 0)


def _dense_layer(p, x, Wl, Wr, b, relu):
    def body(p0_ref, p1_ref, x_ref, wl_ref, wr_ref, b_ref, o_ref):
        agg = p0_ref[...] + p1_ref[...]
        h = jnp.dot(agg, wl_ref[...], preferred_element_type=jnp.float32)
        h = h + jnp.dot(x_ref[...], wr_ref[...], preferred_element_type=jnp.float32)
        h = h + b_ref[...]
        if relu:
            h = jnp.maximum(h, 0.0)
        o_ref[...] = h

    return pl.pallas_call(
        body,
        out_shape=jax.ShapeDtypeStruct((N, D), jnp.float32),
    )(p[0], p[1], x, Wl, Wr, b.reshape(1, D))


def kernel(Features, A, E, Wl1, Wr1, b1, Wl2, Wr2, b2):
    A0, A1 = A[0], A[1]
    E0, E1 = E[0], E[1]
    p1 = _seg_sum(Features, A0, A1)
    h1 = _dense_layer(p1, Features, Wl1, Wr1, b1, True)
    p2 = _seg_sum(h1, A0, A1)
    h2 = _dense_layer(p2, h1, Wl2, Wr2, b2, False)
    return _edge_dot(h2, E0, E1)


# speedup vs baseline: 2.3450x; 2.3450x over previous
import functools

import jax
import jax.numpy as jnp
from jax import lax
from jax.experimental import pallas as pl
from jax.experimental.pallas import tpu as pltpu
from jax.experimental.pallas import tpu_sc as plsc

N = 10000
D = 128
NE = 320000
NC = 2
NS = 16
NW = NC * NS
EPT = NE // NW
C = 80
NCHUNK = EPT // C
NP = 10240
RPT = NP // NS

_mesh = plsc.VectorSubcoreMesh(core_axis_name="c", subcore_axis_name="s")


@functools.partial(
    pl.kernel,
    out_type=jax.ShapeDtypeStruct((NC, NP, D), jnp.float32),
    mesh=_mesh,
    scratch_types=[
        pltpu.VMEM((C,), jnp.int32),
        pltpu.VMEM((C,), jnp.int32),
        pltpu.VMEM((C, D), jnp.float32),
        pltpu.VMEM_SHARED((NP, D), jnp.float32),
        pltpu.SemaphoreType.DMA,
    ],
)
def _seg_sum(x_hbm, src_hbm, dst_hbm, out_hbm, src_v, dst_v, rows_v, acc, sem):
    cid = lax.axis_index("c")
    sid = lax.axis_index("s")
    wid = cid * NS + sid

    def zfill(i, carry):
        for t in range(D // 16):
            rows_v[i, pl.ds(t * 16, 16)] = jnp.zeros((16,), jnp.float32)
        return carry

    lax.fori_loop(0, C, zfill, 0)

    row0 = sid * RPT

    def zcopy(k, carry):
        pltpu.sync_copy(rows_v, acc.at[pl.ds(row0 + k * C, C), :])
        return carry

    lax.fori_loop(0, RPT // C, zcopy, 0)
    plsc.subcore_barrier()

    ebase = wid * EPT

    def chunk(j, carry):
        off = ebase + j * C
        pltpu.sync_copy(src_hbm.at[pl.ds(off, C)], src_v)
        pltpu.sync_copy(dst_hbm.at[pl.ds(off, C)], dst_v)
        pltpu.async_copy(x_hbm.at[src_v], rows_v, sem).wait()
        pltpu.sync_copy(rows_v, acc.at[dst_v], add=True)
        return carry

    lax.fori_loop(0, NCHUNK, chunk, 0)
    plsc.subcore_barrier()

    pltpu.sync_copy(acc.at[pl.ds(row0, RPT), :],
                    out_hbm.at[cid, pl.ds(row0, RPT), :])


@functools.partial(
    pl.kernel,
    out_type=jax.ShapeDtypeStruct((NE,), jnp.float32),
    mesh=_mesh,
    scratch_types=[
        pltpu.VMEM((C,), jnp.int32),
        pltpu.VMEM((C,), jnp.int32),
        pltpu.VMEM((C, D), jnp.float32),
        pltpu.VMEM((C, D), jnp.float32),
        pltpu.VMEM((C,), jnp.float32),
        pltpu.SemaphoreType.DMA,
        pltpu.SemaphoreType.DMA,
    ],
    compiler_params=pltpu.CompilerParams(needs_layout_passes=False),
)
def _edge_dot(h_hbm, e0_hbm, e1_hbm, out_hbm, i0_v, i1_v, r0_v, r1_v, o_v,
              sem0, sem1):
    cid = lax.axis_index("c")
    sid = lax.axis_index("s")
    wid = cid * NS + sid
    ebase = wid * EPT
    lane = lax.iota(jnp.int32, 16)

    def chunk(j, carry):
        off = ebase + j * C
        pltpu.sync_copy(e0_hbm.at[pl.ds(off, C)], i0_v)
        pltpu.sync_copy(e1_hbm.at[pl.ds(off, C)], i1_v)
        d0 = pltpu.async_copy(h_hbm.at[i0_v], r0_v, sem0)
        d1 = pltpu.async_copy(h_hbm.at[i1_v], r1_v, sem1)
        d0.wait()
        d1.wait()

        def group(g, carry2):
            rows = g * 16 + lane

            def dcol(d, acc):
                cols = jnp.full((16,), d, jnp.int32)
                a = plsc.load_gather(r0_v, [rows, cols])
                b = plsc.load_gather(r1_v, [rows, cols])
                return acc + a * b

            acc = lax.fori_loop(0, D, dcol, jnp.zeros((16,), jnp.float32))
            o_v[pl.ds(g * 16, 16)] = 1.0 / (1.0 + jnp.exp(-acc))
            return carry2

        lax.fori_loop(0, C // 16, group, 0)
        pltpu.sync_copy(o_v, out_hbm.at[pl.ds(off, C)])
        return carry

    lax.fori_loop(0, NCHUNK, chunk, 0)


def _dense_layer(p, x, Wl, Wr, b, relu):
    def body(p0_ref, p1_ref, x_ref, wl_ref, wr_ref, b_ref, o_ref):
        agg = p0_ref[:N] + p1_ref[:N]
        h = jnp.dot(agg, wl_ref[...], preferred_element_type=jnp.float32)
        h = h + jnp.dot(x_ref[...], wr_ref[...], preferred_element_type=jnp.float32)
        h = h + b_ref[...]
        if relu:
            h = jnp.maximum(h, 0.0)
        o_ref[...] = h

    return pl.pallas_call(
        body,
        out_shape=jax.ShapeDtypeStruct((N, D), jnp.float32),
    )(p[0], p[1], x, Wl, Wr, b.reshape(1, D))


def kernel(Features, A, E, Wl1, Wr1, b1, Wl2, Wr2, b2):
    A0, A1 = A[0], A[1]
    E0, E1 = E[0], E[1]
    p1 = _seg_sum(Features, A0, A1)
    h1 = _dense_layer(p1, Features, Wl1, Wr1, b1, True)
    p2 = _seg_sum(h1, A0, A1)
    h2 = _dense_layer(p2, h1, Wl2, Wr2, b2, False)
    return _edge_dot(h2, E0, E1)


# speedup vs baseline: 3.4667x; 1.4783x over previous
import functools

import jax
import jax.numpy as jnp
from jax import lax
from jax.experimental import pallas as pl
from jax.experimental.pallas import tpu as pltpu
from jax.experimental.pallas import tpu_sc as plsc

N = 10000
D = 128
NE = 320000
NC = 2
NS = 16
NW = NC * NS
CS = 40
EPT = NE // NW
NCHUNK = EPT // CS
NP = 10240
RPT = NP // NS
C = 80
NCHUNK_E = EPT // C

_mesh = plsc.VectorSubcoreMesh(core_axis_name="c", subcore_axis_name="s")

SB = 8
SLI = SB - 2
SLG = SB - 4


@functools.partial(
    pl.kernel,
    out_type=jax.ShapeDtypeStruct((NC, NP, D), jnp.float32),
    mesh=_mesh,
    scratch_types=(
        [pltpu.VMEM((CS,), jnp.int32) for _ in range(SB)]
        + [pltpu.VMEM((CS,), jnp.int32) for _ in range(SB)]
        + [pltpu.VMEM((CS, D), jnp.float32) for _ in range(SB)]
        + [pltpu.VMEM_SHARED((NP, D), jnp.float32)]
        + [pltpu.SemaphoreType.DMA for _ in range(3 * SB)]
    ),
)
def _seg_sum(x_hbm, src_hbm, dst_hbm, out_hbm, *s):
    srcv = s[0:SB]
    dstv = s[SB:2 * SB]
    rows = s[2 * SB:3 * SB]
    acc = s[3 * SB]
    semi = s[3 * SB + 1:4 * SB + 1]
    semg = s[4 * SB + 1:5 * SB + 1]
    sems = s[5 * SB + 1:6 * SB + 1]

    cid = lax.axis_index("c")
    sid = lax.axis_index("s")
    wid = cid * NS + sid
    ebase = wid * EPT

    def idx_issue(c, b):
        off = ebase + c * CS
        pltpu.async_copy(src_hbm.at[pl.ds(off, CS)], srcv[b], semi[b])
        pltpu.async_copy(dst_hbm.at[pl.ds(off, CS)], dstv[b], semi[b])

    def idx_wait(b):
        pltpu.make_async_copy(src_hbm.at[pl.ds(0, CS)], srcv[b],
                              semi[b]).wait()
        pltpu.make_async_copy(dst_hbm.at[pl.ds(0, CS)], dstv[b],
                              semi[b]).wait()

    def gat_issue(b):
        pltpu.async_copy(x_hbm.at[srcv[b]], rows[b], semg[b])

    def gat_wait(b):
        pltpu.make_async_copy(x_hbm.at[srcv[b]], rows[b], semg[b]).wait()

    def sca_issue(b):
        pltpu.async_copy(rows[b], acc.at[dstv[b]], sems[b], add=True)

    def sca_wait(b):
        pltpu.make_async_copy(rows[b], acc.at[dstv[b]], sems[b]).wait()

    def zfill(i, carry):
        for t in range(D // 16):
            rows[0][i, pl.ds(t * 16, 16)] = jnp.zeros((16,), jnp.float32)
        return carry

    lax.fori_loop(0, CS, zfill, 0)
    row0 = sid * RPT

    def zcopy(k, carry):
        pltpu.sync_copy(rows[0], acc.at[pl.ds(row0 + k * CS, CS), :])
        return carry

    lax.fori_loop(0, RPT // CS, zcopy, 0)
    plsc.subcore_barrier()

    for m in range(SLI):
        idx_issue(m, m % SB)
    for g in range(SLG):
        idx_wait(g % SB)
        gat_issue(g % SB)

    nslots = -(-NCHUNK // SB) * SB

    def outer(j0, carry):
        for b in range(SB):
            j = j0 * SB + b

            @pl.when(j < NCHUNK)
            def _():
                gat_wait(b)
                sca_issue(b)

            m = j + SLI
            bm = (b + SLI) % SB

            @pl.when(jnp.logical_and(m < NCHUNK, j >= SB - SLI))
            def _():
                sca_wait(bm)

            @pl.when(m < NCHUNK)
            def _():
                idx_issue(m, bm)

            g2 = j + SLG
            bg = (b + SLG) % SB

            @pl.when(g2 < NCHUNK)
            def _():
                idx_wait(bg)
                gat_issue(bg)

        return carry

    lax.fori_loop(0, nslots // SB, outer, 0)

    for c in range(max(0, NCHUNK - SB), NCHUNK):
        sca_wait(c % SB)

    plsc.subcore_barrier()
    pltpu.sync_copy(acc.at[pl.ds(row0, RPT), :],
                    out_hbm.at[cid, pl.ds(row0, RPT), :])


EB = 4
ELI = 2
ELG = 1


@functools.partial(
    pl.kernel,
    out_type=jax.ShapeDtypeStruct((NE,), jnp.float32),
    mesh=_mesh,
    scratch_types=(
        [pltpu.VMEM((C,), jnp.int32) for _ in range(EB)]
        + [pltpu.VMEM((C,), jnp.int32) for _ in range(EB)]
        + [pltpu.VMEM((C, D), jnp.float32) for _ in range(EB)]
        + [pltpu.VMEM((C, D), jnp.float32) for _ in range(EB)]
        + [pltpu.VMEM((C,), jnp.float32) for _ in range(EB)]
        + [pltpu.SemaphoreType.DMA for _ in range(3 * EB)]
    ),
    compiler_params=pltpu.CompilerParams(needs_layout_passes=False),
)
def _edge_dot(h_hbm, e0_hbm, e1_hbm, out_hbm, *s):
    i0 = s[0:EB]
    i1 = s[EB:2 * EB]
    r0 = s[2 * EB:3 * EB]
    r1 = s[3 * EB:4 * EB]
    ov = s[4 * EB:5 * EB]
    semi = s[5 * EB:6 * EB]
    semg = s[6 * EB:7 * EB]
    semo = s[7 * EB:8 * EB]

    cid = lax.axis_index("c")
    sid = lax.axis_index("s")
    wid = cid * NS + sid
    ebase = wid * EPT
    lane = lax.iota(jnp.int32, 16)

    def idx_issue(c, b):
        off = ebase + c * C
        pltpu.async_copy(e0_hbm.at[pl.ds(off, C)], i0[b], semi[b])
        pltpu.async_copy(e1_hbm.at[pl.ds(off, C)], i1[b], semi[b])

    def idx_wait(b):
        pltpu.make_async_copy(e0_hbm.at[pl.ds(0, C)], i0[b], semi[b]).wait()
        pltpu.make_async_copy(e1_hbm.at[pl.ds(0, C)], i1[b], semi[b]).wait()

    def gat_issue(b):
        pltpu.async_copy(h_hbm.at[i0[b]], r0[b], semg[b])
        pltpu.async_copy(h_hbm.at[i1[b]], r1[b], semg[b])

    def gat_wait(b):
        pltpu.make_async_copy(h_hbm.at[i0[b]], r0[b], semg[b]).wait()
        pltpu.make_async_copy(h_hbm.at[i1[b]], r1[b], semg[b]).wait()

    def out_issue(c, b):
        off = ebase + c * C
        pltpu.async_copy(ov[b], out_hbm.at[pl.ds(off, C)], semo[b])

    def out_wait(b):
        pltpu.make_async_copy(ov[b], out_hbm.at[pl.ds(0, C)], semo[b]).wait()

    def compute(b):
        def group(g, carry):
            rows16 = g * 16 + lane
            acc = jnp.zeros((16,), jnp.float32)
            for d in range(D):
                cols = jnp.full((16,), d, jnp.int32)
                a = plsc.load_gather(r0[b], [rows16, cols])
                v = plsc.load_gather(r1[b], [rows16, cols])
                acc = acc + a * v
            ov[b][pl.ds(g * 16, 16)] = 1.0 / (1.0 + jnp.exp(-acc))
            return carry

        lax.fori_loop(0, C // 16, group, 0)

    for m in range(ELI):
        idx_issue(m, m % EB)
    for g in range(ELG):
        idx_wait(g % EB)
        gat_issue(g % EB)

    nslots = -(-NCHUNK_E // EB) * EB

    def outer(j0, carry):
        for b in range(EB):
            j = j0 * EB + b

            m = j + ELI
            bm = (b + ELI) % EB

            @pl.when(jnp.logical_and(m < NCHUNK_E, j >= EB - ELI))
            def _():
                out_wait(bm)

            @pl.when(m < NCHUNK_E)
            def _():
                idx_issue(m, bm)

            g2 = j + ELG
            bg = (b + ELG) % EB

            @pl.when(g2 < NCHUNK_E)
            def _():
                idx_wait(bg)
                gat_issue(bg)

            @pl.when(j < NCHUNK_E)
            def _():
                gat_wait(b)
                compute(b)
                out_issue(j, b)

        return carry

    lax.fori_loop(0, nslots // EB, outer, 0)

    for c in range(max(0, NCHUNK_E - EB), NCHUNK_E):
        out_wait(c % EB)


def _dense(p, x, Wl, Wr, b, relu):
    def body(p_ref, x_ref, wl_ref, wr_ref, b_ref, o_ref):
        agg = p_ref[0, :N] + p_ref[1, :N]
        h = jnp.dot(agg, wl_ref[...], preferred_element_type=jnp.float32)
        h = h + jnp.dot(x_ref[...], wr_ref[...],
                        preferred_element_type=jnp.float32)
        h = h + b_ref[...]
        if relu:
            h = jnp.maximum(h, 0.0)
        o_ref[...] = h

    return pl.pallas_call(
        body,
        out_shape=jax.ShapeDtypeStruct((N, D), jnp.float32),
    )(p, x, Wl, Wr, b.reshape(1, D))


def kernel(Features, A, E, Wl1, Wr1, b1, Wl2, Wr2, b2):
    A0, A1 = A[0], A[1]
    E0, E1 = E[0], E[1]
    p1 = _seg_sum(Features, A0, A1)
    h1 = _dense(p1, Features, Wl1, Wr1, b1, True)
    p2 = _seg_sum(h1, A0, A1)
    h2 = _dense(p2, h1, Wl2, Wr2, b2, False)
    return _edge_dot(h2, E0, E1)
